# baseline (speedup 1.0000x reference)
import functools
import jax
import jax.numpy as jnp
from jax import lax
from jax.experimental import pallas as pl
from jax.experimental.pallas import tpu as pltpu
from jax.experimental.pallas import tpu_sc as plsc

NSEG = 8
CHUNK = 128
LANES = 16


def _take16(v, idx):
    dnums = lax.GatherDimensionNumbers(
        offset_dims=(), collapsed_slice_dims=(0,), start_index_map=(0,))
    return lax.gather(v, idx[:, None], dnums, slice_sizes=(1,),
                      mode=lax.GatherScatterMode.PROMISE_IN_BOUNDS)


def _rsqrt16(x):
    i = lax.bitcast_convert_type(x, jnp.int32)
    i = jnp.int32(0x5F3759DF) - lax.shift_right_arithmetic(i, 1)
    y = lax.bitcast_convert_type(i, jnp.float32)
    for _ in range(3):
        y = y * (1.5 - 0.5 * x * y * y)
    return y


def _sc_segment_sums(x, nt_vec):
    B, L, D = x.shape
    ND = D // LANES
    mesh = plsc.VectorSubcoreMesh(core_axis_name="c", subcore_axis_name="s")

    @functools.partial(
        pl.kernel,
        mesh=mesh,
        out_type=jax.ShapeDtypeStruct((B, 2, NSEG, D), jnp.float32),
        scratch_types=[
            pltpu.VMEM((LANES,), jnp.int32),
            pltpu.VMEM((CHUNK, D), jnp.float32),
            pltpu.VMEM((NSEG, D), jnp.float32),
        ],
    )
    def k(x_hbm, nt_hbm, out_hbm, nt_v, buf_v, part_v):
        cid = lax.axis_index("c")
        sid = lax.axis_index("s")
        wid = sid * 2 + cid
        b = wid // 2
        half = wid - b * 2

        pltpu.sync_copy(nt_hbm, nt_v)
        nt = nt_v[pl.ds(0, LANES)][0]
        step = nt // NSEG
        h = (nt + 1) // 2
        lo = half * h
        hi = jnp.minimum(nt, lo + h)

        zero = jnp.zeros((LANES,), jnp.float32)
        for r in range(NSEG):
            for dd in range(ND):
                part_v[r, pl.ds(dd * LANES, LANES)] = zero

        lo8 = (lo // 8) * 8
        cnt = hi - lo8
        nchunks = lax.max((cnt + CHUNK - 1) // CHUNK, 0)

        def chunk_body(kk, _):
            start = lo8 + kk * CHUNK
            startc = jnp.minimum(start, L - CHUNK)
            pltpu.sync_copy(x_hbm.at[b, pl.ds(startc, CHUNK), :], buf_v)
            t0 = jnp.maximum(lo, start)
            i0 = t0 - startc
            i1 = jnp.minimum(hi, start + CHUNK) - startc

            seg0 = jnp.int32(0)
            for si in range(1, NSEG):
                seg0 = seg0 + (t0 >= si * step).astype(jnp.int32)
            seg0 = jnp.minimum(seg0, NSEG - 1)
            nb0 = jnp.where(seg0 >= NSEG - 1, nt, (seg0 + 1) * step)

            def tok_body(i, carry):
                seg, nb = carry
                t = startc + i
                cross = (t >= nb).astype(jnp.int32)
                seg = jnp.minimum(seg + cross, NSEG - 1)
                nb = jnp.where(cross == 1,
                               jnp.where(seg >= NSEG - 1, nt, (seg + 1) * step),
                               nb)
                vs = [buf_v[i, pl.ds(dd * LANES, LANES)] for dd in range(ND)]
                ss = vs[0] * vs[0]
                for dd in range(1, ND):
                    ss = ss + vs[dd] * vs[dd]
                ii = lax.broadcasted_iota(jnp.int32, (LANES,), 0)
                for kbit in (1, 2, 4, 8):
                    ss = ss + _take16(ss, ii ^ kbit)
                rs = _rsqrt16(ss)
                for dd in range(ND):
                    plsc.addupdate(part_v.at[seg, pl.ds(dd * LANES, LANES)],
                                   vs[dd] * rs)
                return (seg, nb)

            lax.fori_loop(i0, i1, tok_body, (seg0, nb0))
            return 0

        lax.fori_loop(0, nchunks, chunk_body, 0)
        pltpu.sync_copy(part_v, out_hbm.at[b, half])

    return k(x, nt_vec)


def _classify_kernel(nt_ref, p_ref, e_ref, preds_ref):
    nt = nt_ref[0]
    step = nt // NSEG
    e = e_ref[...]
    en = e / jnp.sqrt(jnp.sum(e * e, axis=-1, keepdims=True))
    acc = p_ref[:, 0, :, :] + p_ref[:, 1, :, :]
    B = acc.shape[0]
    acc2 = acc.reshape(B * NSEG, acc.shape[-1])
    srow = jax.lax.broadcasted_iota(jnp.int32, (B * NSEG, 1), 0) % NSEG
    denom = jnp.where(srow == NSEG - 1,
                      nt - (NSEG - 1) * step, step).astype(jnp.float32)
    seg = acc2 / denom
    segn = seg / jnp.sqrt(jnp.sum(seg * seg, axis=-1, keepdims=True))
    cos = jax.lax.dot_general(segn, en, (((1,), (1,)), ((), ())),
                              preferred_element_type=jnp.float32)
    mx = jnp.max(cos, axis=-1, keepdims=True)
    idx = jax.lax.broadcasted_iota(jnp.int32, cos.shape, 1)
    pick = jnp.min(jnp.where(cos >= mx, idx, cos.shape[-1]), axis=-1,
                   keepdims=True)
    preds_ref[...] = pick.astype(jnp.float32)


def kernel(x, num_tokens, emotion_embs):
    B, L, D = x.shape
    E = emotion_embs.shape[0]
    nt32 = num_tokens.astype(jnp.int32)
    nt_vec = jnp.tile(nt32, LANES)
    partials = _sc_segment_sums(x, nt_vec)
    preds = pl.pallas_call(
        _classify_kernel,
        grid_spec=pltpu.PrefetchScalarGridSpec(
            num_scalar_prefetch=1,
            grid=(1,),
            in_specs=[
                pl.BlockSpec((B, 2, NSEG, D), lambda i, nt_ref: (0, 0, 0, 0)),
                pl.BlockSpec((E, D), lambda i, nt_ref: (0, 0)),
            ],
            out_specs=pl.BlockSpec((B * NSEG, 1), lambda i, nt_ref: (0, 0)),
        ),
        out_shape=jax.ShapeDtypeStruct((B * NSEG, 1), jnp.float32),
    )(nt32, partials, emotion_embs)
    return preds.reshape(B, NSEG)


# speedup vs baseline: 1.0230x; 1.0230x over previous
import functools
import jax
import jax.numpy as jnp
from jax import lax
from jax.experimental import pallas as pl
from jax.experimental.pallas import tpu as pltpu
from jax.experimental.pallas import tpu_sc as plsc

NSEG = 8
CHUNK = 128
LANES = 16


def _take16(v, idx):
    dnums = lax.GatherDimensionNumbers(
        offset_dims=(), collapsed_slice_dims=(0,), start_index_map=(0,))
    return lax.gather(v, idx[:, None], dnums, slice_sizes=(1,),
                      mode=lax.GatherScatterMode.PROMISE_IN_BOUNDS)


def _rsqrt16(x):
    i = lax.bitcast_convert_type(x, jnp.int32)
    i = jnp.int32(0x5F3759DF) - lax.shift_right_arithmetic(i, 1)
    y = lax.bitcast_convert_type(i, jnp.float32)
    for _ in range(3):
        y = y * (1.5 - 0.5 * x * y * y)
    return y


def _sc_segment_sums(x, nt_vec):
    B, L, D = x.shape
    ND = D // LANES
    mesh = plsc.VectorSubcoreMesh(core_axis_name="c", subcore_axis_name="s")

    @functools.partial(
        pl.kernel,
        mesh=mesh,
        out_type=jax.ShapeDtypeStruct((B, 2, NSEG, D), jnp.float32),
        scratch_types=[
            pltpu.VMEM((LANES,), jnp.int32),
            pltpu.VMEM((CHUNK, D), jnp.float32),
            pltpu.VMEM((NSEG, D), jnp.float32),
        ],
    )
    def k(x_hbm, nt_hbm, out_hbm, nt_v, buf_v, part_v):
        cid = lax.axis_index("c")
        sid = lax.axis_index("s")
        wid = sid * 2 + cid
        b = wid // 2
        half = wid - b * 2

        pltpu.sync_copy(nt_hbm, nt_v)
        nt = nt_v[pl.ds(0, LANES)][0]
        step = nt // NSEG
        h = (nt + 1) // 2
        lo = half * h
        hi = jnp.minimum(nt, lo + h)

        zero = jnp.zeros((LANES,), jnp.float32)
        for r in range(NSEG):
            for dd in range(ND):
                part_v[r, pl.ds(dd * LANES, LANES)] = zero

        lo8 = (lo // 8) * 8
        cnt = hi - lo8
        nchunks = lax.max((cnt + CHUNK - 1) // CHUNK, 0)

        def chunk_body(kk, _):
            start = lo8 + kk * CHUNK
            startc = jnp.minimum(start, L - CHUNK)
            pltpu.sync_copy(x_hbm.at[b, pl.ds(startc, CHUNK), :], buf_v)
            t0 = jnp.maximum(lo, start)
            i0 = t0 - startc
            i1 = jnp.minimum(hi, start + CHUNK) - startc
            ii = lax.broadcasted_iota(jnp.int32, (LANES,), 0)

            def one_token(i_raw):
                i = jnp.minimum(i_raw, i1 - 1)
                t = startc + i
                seg = jnp.int32(0)
                for si in range(1, NSEG):
                    seg = seg + (t >= si * step).astype(jnp.int32)
                seg = jnp.minimum(seg, NSEG - 1)
                vs = [buf_v[i, pl.ds(dd * LANES, LANES)] for dd in range(ND)]
                sq = [vs[dd] * vs[dd] for dd in range(ND)]
                while len(sq) > 1:
                    sq = [sq[2 * p] + sq[2 * p + 1] for p in range(len(sq) // 2)]
                ss = sq[0]
                for kbit in (1, 2, 4, 8):
                    ss = ss + _take16(ss, ii ^ kbit)
                rs = _rsqrt16(ss)
                rs = rs * (i_raw < i1).astype(jnp.float32)
                for dd in range(ND):
                    plsc.addupdate(part_v.at[seg, pl.ds(dd * LANES, LANES)],
                                   vs[dd] * rs)

            def quad_body(u, _):
                base = i0 + 4 * u
                for j in range(4):
                    one_token(base + j)
                return 0

            nquads = lax.max((i1 - i0 + 3) // 4, 0)
            lax.fori_loop(0, nquads, quad_body, 0)
            return 0

        lax.fori_loop(0, nchunks, chunk_body, 0)
        pltpu.sync_copy(part_v, out_hbm.at[b, half])

    return k(x, nt_vec)


def _classify_kernel(nt_ref, p_ref, e_ref, preds_ref):
    nt = nt_ref[0]
    step = nt // NSEG
    e = e_ref[...]
    en = e / jnp.sqrt(jnp.sum(e * e, axis=-1, keepdims=True))
    acc = p_ref[:, 0, :, :] + p_ref[:, 1, :, :]
    B = acc.shape[0]
    acc2 = acc.reshape(B * NSEG, acc.shape[-1])
    srow = jax.lax.broadcasted_iota(jnp.int32, (B * NSEG, 1), 0) % NSEG
    denom = jnp.where(srow == NSEG - 1,
                      nt - (NSEG - 1) * step, step).astype(jnp.float32)
    seg = acc2 / denom
    segn = seg / jnp.sqrt(jnp.sum(seg * seg, axis=-1, keepdims=True))
    cos = jax.lax.dot_general(segn, en, (((1,), (1,)), ((), ())),
                              preferred_element_type=jnp.float32)
    mx = jnp.max(cos, axis=-1, keepdims=True)
    idx = jax.lax.broadcasted_iota(jnp.int32, cos.shape, 1)
    pick = jnp.min(jnp.where(cos >= mx, idx, cos.shape[-1]), axis=-1,
                   keepdims=True)
    preds_ref[...] = pick.astype(jnp.float32)


def kernel(x, num_tokens, emotion_embs):
    B, L, D = x.shape
    E = emotion_embs.shape[0]
    nt32 = num_tokens.astype(jnp.int32)
    nt_vec = jnp.tile(nt32, LANES)
    partials = _sc_segment_sums(x, nt_vec)
    preds = pl.pallas_call(
        _classify_kernel,
        grid_spec=pltpu.PrefetchScalarGridSpec(
            num_scalar_prefetch=1,
            grid=(1,),
            in_specs=[
                pl.BlockSpec((B, 2, NSEG, D), lambda i, nt_ref: (0, 0, 0, 0)),
                pl.BlockSpec((E, D), lambda i, nt_ref: (0, 0)),
            ],
            out_specs=pl.BlockSpec((B * NSEG, 1), lambda i, nt_ref: (0, 0)),
        ),
        out_shape=jax.ShapeDtypeStruct((B * NSEG, 1), jnp.float32),
    )(nt32, partials, emotion_embs)
    return preds.reshape(B, NSEG)


# speedup vs baseline: 1.0337x; 1.0104x over previous
import functools
import jax
import jax.numpy as jnp
from jax import lax
from jax.experimental import pallas as pl
from jax.experimental.pallas import tpu as pltpu
from jax.experimental.pallas import tpu_sc as plsc

NSEG = 8
CHUNK = 128
LANES = 16


def _take16(v, idx):
    dnums = lax.GatherDimensionNumbers(
        offset_dims=(), collapsed_slice_dims=(0,), start_index_map=(0,))
    return lax.gather(v, idx[:, None], dnums, slice_sizes=(1,),
                      mode=lax.GatherScatterMode.PROMISE_IN_BOUNDS)


def _rsqrt16(x):
    i = lax.bitcast_convert_type(x, jnp.int32)
    i = jnp.int32(0x5F3759DF) - lax.shift_right_arithmetic(i, 1)
    y = lax.bitcast_convert_type(i, jnp.float32)
    for _ in range(3):
        y = y * (1.5 - 0.5 * x * y * y)
    return y


def _sc_segment_sums(x, nt_vec):
    B, L, D = x.shape
    ND = D // LANES
    mesh = plsc.VectorSubcoreMesh(core_axis_name="c", subcore_axis_name="s")

    @functools.partial(
        pl.kernel,
        mesh=mesh,
        out_type=jax.ShapeDtypeStruct((B, 2, NSEG, D), jnp.float32),
        scratch_types=[
            pltpu.VMEM((LANES,), jnp.int32),
            pltpu.VMEM((CHUNK, D), jnp.float32),
            pltpu.VMEM((NSEG, D), jnp.float32),
        ],
    )
    def k(x_hbm, nt_hbm, out_hbm, nt_v, buf_v, part_v):
        cid = lax.axis_index("c")
        sid = lax.axis_index("s")
        wid = sid * 2 + cid
        b = wid // 2
        half = wid - b * 2

        pltpu.sync_copy(nt_hbm, nt_v)
        nt = nt_v[pl.ds(0, LANES)][0]
        step = nt // NSEG
        h = (nt + 1) // 2
        lo = half * h
        hi = jnp.minimum(nt, lo + h)

        zero = jnp.zeros((LANES,), jnp.float32)
        for r in range(NSEG):
            for dd in range(ND):
                part_v[r, pl.ds(dd * LANES, LANES)] = zero

        lo8 = (lo // 8) * 8
        cnt = hi - lo8
        nchunks = lax.max((cnt + CHUNK - 1) // CHUNK, 0)

        def chunk_body(kk, _):
            start = lo8 + kk * CHUNK
            startc = jnp.minimum(start, L - CHUNK)
            pltpu.sync_copy(x_hbm.at[b, pl.ds(startc, CHUNK), :], buf_v)
            t0 = jnp.maximum(lo, start)
            i0 = t0 - startc
            i1 = jnp.minimum(hi, start + CHUNK) - startc
            ii = lax.broadcasted_iota(jnp.int32, (LANES,), 0)

            def one_token(i_raw, b_hi):
                i = jnp.minimum(i_raw, b_hi - 1)
                vs = [buf_v[i, pl.ds(dd * LANES, LANES)] for dd in range(ND)]
                sq = [vs[dd] * vs[dd] for dd in range(ND)]
                while len(sq) > 1:
                    sq = [sq[2 * p] + sq[2 * p + 1] for p in range(len(sq) // 2)]
                ss = sq[0]
                for kbit in (1, 2, 4, 8):
                    ss = ss + _take16(ss, ii ^ kbit)
                rs = _rsqrt16(ss)
                rs = rs * (i_raw < b_hi).astype(jnp.float32)
                return [vs[dd] * rs for dd in range(ND)]

            for s in range(NSEG):
                s_lo = s * step
                s_hi = nt if s == NSEG - 1 else (s + 1) * step
                a = jnp.maximum(i0, s_lo - startc)
                bb = jnp.minimum(i1, s_hi - startc)

                @pl.when(bb > a)
                def _run(s=s, a=a, bb=bb):
                    zero = jnp.zeros((LANES,), jnp.float32)

                    def pair_body(u, acc):
                        base = a + 2 * u
                        c0 = one_token(base, bb)
                        c1 = one_token(base + 1, bb)
                        return tuple(acc[dd] + (c0[dd] + c1[dd])
                                     for dd in range(ND))

                    npairs = (bb - a + 1) // 2
                    acc = lax.fori_loop(0, npairs, pair_body,
                                        tuple(zero for _ in range(ND)))
                    for dd in range(ND):
                        plsc.addupdate(part_v.at[s, pl.ds(dd * LANES, LANES)],
                                       acc[dd])
            return 0

        lax.fori_loop(0, nchunks, chunk_body, 0)
        pltpu.sync_copy(part_v, out_hbm.at[b, half])

    return k(x, nt_vec)


def _classify_kernel(nt_ref, p_ref, e_ref, preds_ref):
    nt = nt_ref[0]
    step = nt // NSEG
    e = e_ref[...]
    en = e / jnp.sqrt(jnp.sum(e * e, axis=-1, keepdims=True))
    acc = p_ref[:, 0, :, :] + p_ref[:, 1, :, :]
    B = acc.shape[0]
    acc2 = acc.reshape(B * NSEG, acc.shape[-1])
    srow = jax.lax.broadcasted_iota(jnp.int32, (B * NSEG, 1), 0) % NSEG
    denom = jnp.where(srow == NSEG - 1,
                      nt - (NSEG - 1) * step, step).astype(jnp.float32)
    seg = acc2 / denom
    segn = seg / jnp.sqrt(jnp.sum(seg * seg, axis=-1, keepdims=True))
    cos = jax.lax.dot_general(segn, en, (((1,), (1,)), ((), ())),
                              preferred_element_type=jnp.float32)
    mx = jnp.max(cos, axis=-1, keepdims=True)
    idx = jax.lax.broadcasted_iota(jnp.int32, cos.shape, 1)
    pick = jnp.min(jnp.where(cos >= mx, idx, cos.shape[-1]), axis=-1,
                   keepdims=True)
    preds_ref[...] = pick.astype(jnp.float32)


def kernel(x, num_tokens, emotion_embs):
    B, L, D = x.shape
    E = emotion_embs.shape[0]
    nt32 = num_tokens.astype(jnp.int32)
    nt_vec = jnp.tile(nt32, LANES)
    partials = _sc_segment_sums(x, nt_vec)
    preds = pl.pallas_call(
        _classify_kernel,
        grid_spec=pltpu.PrefetchScalarGridSpec(
            num_scalar_prefetch=1,
            grid=(1,),
            in_specs=[
                pl.BlockSpec((B, 2, NSEG, D), lambda i, nt_ref: (0, 0, 0, 0)),
                pl.BlockSpec((E, D), lambda i, nt_ref: (0, 0)),
            ],
            out_specs=pl.BlockSpec((B * NSEG, 1), lambda i, nt_ref: (0, 0)),
        ),
        out_shape=jax.ShapeDtypeStruct((B * NSEG, 1), jnp.float32),
    )(nt32, partials, emotion_embs)
    return preds.reshape(B, NSEG)


# speedup vs baseline: 1.0717x; 1.0367x over previous
import functools
import jax
import jax.numpy as jnp
from jax import lax
from jax.experimental import pallas as pl
from jax.experimental.pallas import tpu as pltpu
from jax.experimental.pallas import tpu_sc as plsc

NSEG = 8
CHUNK = 256
LANES = 16


def _take16(v, idx):
    dnums = lax.GatherDimensionNumbers(
        offset_dims=(), collapsed_slice_dims=(0,), start_index_map=(0,))
    return lax.gather(v, idx[:, None], dnums, slice_sizes=(1,),
                      mode=lax.GatherScatterMode.PROMISE_IN_BOUNDS)


def _rsqrt16(x):
    i = lax.bitcast_convert_type(x, jnp.int32)
    i = jnp.int32(0x5F3759DF) - lax.shift_right_arithmetic(i, 1)
    y = lax.bitcast_convert_type(i, jnp.float32)
    for _ in range(3):
        y = y * (1.5 - 0.5 * x * y * y)
    return y


def _sc_segment_sums(x, nt_vec):
    B, L, D = x.shape
    ND = D // LANES
    mesh = plsc.VectorSubcoreMesh(core_axis_name="c", subcore_axis_name="s")

    @functools.partial(
        pl.kernel,
        mesh=mesh,
        out_type=jax.ShapeDtypeStruct((B, 2, NSEG, D), jnp.float32),
        scratch_types=[
            pltpu.VMEM((LANES,), jnp.int32),
            pltpu.VMEM((CHUNK, D), jnp.float32),
            pltpu.VMEM((NSEG, D), jnp.float32),
        ],
    )
    def k(x_hbm, nt_hbm, out_hbm, nt_v, buf_v, part_v):
        cid = lax.axis_index("c")
        sid = lax.axis_index("s")
        wid = sid * 2 + cid
        b = wid // 2
        half = wid - b * 2

        pltpu.sync_copy(nt_hbm, nt_v)
        nt = nt_v[pl.ds(0, LANES)][0]
        step = nt // NSEG
        h = (nt + 1) // 2
        lo = half * h
        hi = jnp.minimum(nt, lo + h)

        zero = jnp.zeros((LANES,), jnp.float32)
        for r in range(NSEG):
            for dd in range(ND):
                part_v[r, pl.ds(dd * LANES, LANES)] = zero

        lo8 = (lo // 8) * 8
        cnt = hi - lo8
        nchunks = lax.max((cnt + CHUNK - 1) // CHUNK, 0)

        def chunk_body(kk, _):
            start = lo8 + kk * CHUNK
            startc = jnp.minimum(start, L - CHUNK)
            pltpu.sync_copy(x_hbm.at[b, pl.ds(startc, CHUNK), :], buf_v)
            t0 = jnp.maximum(lo, start)
            i0 = t0 - startc
            i1 = jnp.minimum(hi, start + CHUNK) - startc
            ii = lax.broadcasted_iota(jnp.int32, (LANES,), 0)

            def one_token(i_raw, b_hi):
                i = jnp.minimum(i_raw, b_hi - 1)
                vs = [buf_v[i, pl.ds(dd * LANES, LANES)] for dd in range(ND)]
                sq = [vs[dd] * vs[dd] for dd in range(ND)]
                while len(sq) > 1:
                    sq = [sq[2 * p] + sq[2 * p + 1] for p in range(len(sq) // 2)]
                ss = sq[0]
                for kbit in (1, 2, 4, 8):
                    ss = ss + _take16(ss, ii ^ kbit)
                rs = _rsqrt16(ss)
                rs = rs * (i_raw < b_hi).astype(jnp.float32)
                return [vs[dd] * rs for dd in range(ND)]

            for s in range(NSEG):
                s_lo = s * step
                s_hi = nt if s == NSEG - 1 else (s + 1) * step
                a = jnp.maximum(i0, s_lo - startc)
                bb = jnp.minimum(i1, s_hi - startc)

                @pl.when(bb > a)
                def _run(s=s, a=a, bb=bb):
                    zero = jnp.zeros((LANES,), jnp.float32)

                    def pair_body(u, acc):
                        base = a + 2 * u
                        c0 = one_token(base, bb)
                        c1 = one_token(base + 1, bb)
                        return tuple(acc[dd] + (c0[dd] + c1[dd])
                                     for dd in range(ND))

                    npairs = (bb - a + 1) // 2
                    acc = lax.fori_loop(0, npairs, pair_body,
                                        tuple(zero for _ in range(ND)))
                    for dd in range(ND):
                        plsc.addupdate(part_v.at[s, pl.ds(dd * LANES, LANES)],
                                       acc[dd])
            return 0

        lax.fori_loop(0, nchunks, chunk_body, 0)
        pltpu.sync_copy(part_v, out_hbm.at[b, half])

    return k(x, nt_vec)


def _classify_kernel(nt_ref, p_ref, e_ref, preds_ref):
    nt = nt_ref[0]
    step = nt // NSEG
    e = e_ref[...]
    en = e / jnp.sqrt(jnp.sum(e * e, axis=-1, keepdims=True))
    acc = p_ref[:, 0, :, :] + p_ref[:, 1, :, :]
    B = acc.shape[0]
    acc2 = acc.reshape(B * NSEG, acc.shape[-1])
    srow = jax.lax.broadcasted_iota(jnp.int32, (B * NSEG, 1), 0) % NSEG
    denom = jnp.where(srow == NSEG - 1,
                      nt - (NSEG - 1) * step, step).astype(jnp.float32)
    seg = acc2 / denom
    segn = seg / jnp.sqrt(jnp.sum(seg * seg, axis=-1, keepdims=True))
    cos = jax.lax.dot_general(segn, en, (((1,), (1,)), ((), ())),
                              preferred_element_type=jnp.float32)
    mx = jnp.max(cos, axis=-1, keepdims=True)
    idx = jax.lax.broadcasted_iota(jnp.int32, cos.shape, 1)
    pick = jnp.min(jnp.where(cos >= mx, idx, cos.shape[-1]), axis=-1,
                   keepdims=True)
    preds_ref[...] = pick.astype(jnp.float32)


def kernel(x, num_tokens, emotion_embs):
    B, L, D = x.shape
    E = emotion_embs.shape[0]
    nt32 = num_tokens.astype(jnp.int32)
    nt_vec = jnp.tile(nt32, LANES)
    partials = _sc_segment_sums(x, nt_vec)
    preds = pl.pallas_call(
        _classify_kernel,
        grid_spec=pltpu.PrefetchScalarGridSpec(
            num_scalar_prefetch=1,
            grid=(1,),
            in_specs=[
                pl.BlockSpec((B, 2, NSEG, D), lambda i, nt_ref: (0, 0, 0, 0)),
                pl.BlockSpec((E, D), lambda i, nt_ref: (0, 0)),
            ],
            out_specs=pl.BlockSpec((B * NSEG, 1), lambda i, nt_ref: (0, 0)),
        ),
        out_shape=jax.ShapeDtypeStruct((B * NSEG, 1), jnp.float32),
    )(nt32, partials, emotion_embs)
    return preds.reshape(B, NSEG)
